# baseline (speedup 1.0000x reference)
import functools

import jax
import jax.numpy as jnp
from jax import lax
from jax.experimental import pallas as pl
from jax.experimental.pallas import tpu as pltpu
from jax.experimental.pallas import tpu_sc as plsc

VOCAB = 1000
D = 1000
DP = 1024
B, T = 4, 2048
NSPLIT = 4
TH = T // NSPLIT
NC, NS = 2, 16
NW = NC * NS
B_PER_W = (B * TH) // NW
CHUNK = 32
NBUF = 3
NCHUNKS = B_PER_W // CHUNK
W_PER_B = TH // B_PER_W

_MESH = plsc.VectorSubcoreMesh(
    core_axis_name="c", subcore_axis_name="s", num_cores=NC, num_subcores=NS)


@functools.partial(
    pl.kernel,
    out_type=jax.ShapeDtypeStruct((B, TH, DP), jnp.float32),
    mesh=_MESH,
    scratch_types=[
        pltpu.VMEM((B_PER_W,), jnp.int32),
        pltpu.VMEM((NBUF, CHUNK, DP), jnp.float32),
        pltpu.SemaphoreType.DMA,
        pltpu.SemaphoreType.DMA,
    ],
)
def _gather_rows(idx_hbm, table_hbm, out_hbm, idx_v, rows_v, gsem, ssem):
    wid = lax.axis_index("s") * NC + lax.axis_index("c")
    b = wid // W_PER_B
    t0 = (wid % W_PER_B) * B_PER_W
    pltpu.sync_copy(idx_hbm.at[pl.ds(wid * B_PER_W, B_PER_W)], idx_v)

    gathers = [None] * NCHUNKS
    scatters = [None] * NCHUNKS
    s_waited = [False] * NCHUNKS
    for ch in range(min(NBUF, NCHUNKS)):
        gathers[ch] = pltpu.async_copy(
            table_hbm.at[idx_v.at[pl.ds(ch * CHUNK, CHUNK)]],
            rows_v.at[ch], gsem)
    for ch in range(NCHUNKS):
        gathers[ch].wait()
        scatters[ch] = pltpu.async_copy(
            rows_v.at[ch % NBUF],
            out_hbm.at[b, pl.ds(t0 + ch * CHUNK, CHUNK)], ssem)
        prev = ch - (NBUF - 1)
        nxt = ch + 1
        if prev >= 0 and nxt < NCHUNKS and gathers[nxt] is None:
            scatters[prev].wait()
            s_waited[prev] = True
            gathers[nxt] = pltpu.async_copy(
                table_hbm.at[idx_v.at[pl.ds(nxt * CHUNK, CHUNK)]],
                rows_v.at[nxt % NBUF], gsem)
    for ch in range(NCHUNKS):
        if not s_waited[ch]:
            scatters[ch].wait()


_TBLK = 512


def _transpose_body(half_ref, out_ref):
    t = half_ref[0].T
    out_ref[...] = t[None, :D, :]


def _transpose_body_acc(acc_ref, half_ref, out_ref):
    del acc_ref
    _transpose_body(half_ref, out_ref)


def _transpose_into(acc, half, h):
    grid = (B, TH // _TBLK)
    out_block = pl.BlockSpec(
        (1, D, _TBLK), lambda b, t, h=h: (b, 0, h * (TH // _TBLK) + t))
    half_block = pl.BlockSpec((1, _TBLK, DP), lambda b, t: (b, t, 0))
    if acc is None:
        return pl.pallas_call(
            _transpose_body,
            grid=grid,
            in_specs=[half_block],
            out_specs=out_block,
            out_shape=jax.ShapeDtypeStruct((B, D, T), jnp.float32),
        )(half)
    return pl.pallas_call(
        _transpose_body_acc,
        grid=grid,
        in_specs=[pl.BlockSpec(memory_space=pltpu.MemorySpace.HBM), half_block],
        out_specs=out_block,
        out_shape=jax.ShapeDtypeStruct((B, D, T), jnp.float32),
        input_output_aliases={0: 0},
    )(acc, half)


def kernel(idx, targets, table):
    del targets
    if idx.dtype != jnp.int32:
        idx = idx.astype(jnp.int32)
    table_pad = jnp.pad(table, ((0, 0), (0, DP - D)))
    halves = [
        _gather_rows(idx[:, h * TH:(h + 1) * TH].reshape(B * TH), table_pad)
        for h in range(NSPLIT)
    ]
    acc = None
    for h, half in enumerate(halves):
        acc = _transpose_into(acc, half, h)
    return acc.transpose(0, 2, 1)


# speedup vs baseline: 1.0412x; 1.0412x over previous
import functools

import jax
import jax.numpy as jnp
from jax import lax
from jax.experimental import pallas as pl
from jax.experimental.pallas import tpu as pltpu
from jax.experimental.pallas import tpu_sc as plsc

VOCAB = 1000
D = 1000
DP = 1024
B, T = 4, 2048
NSPLIT = 2
TH = T // NSPLIT
NC, NS = 2, 16
NW = NC * NS
B_PER_W = (B * TH) // NW
CHUNK = 32
NBUF = 3
NCHUNKS = B_PER_W // CHUNK
W_PER_B = TH // B_PER_W

_MESH = plsc.VectorSubcoreMesh(
    core_axis_name="c", subcore_axis_name="s", num_cores=NC, num_subcores=NS)


@functools.partial(
    pl.kernel,
    out_type=jax.ShapeDtypeStruct((B, TH, DP), jnp.float32),
    mesh=_MESH,
    scratch_types=[
        pltpu.VMEM((B_PER_W,), jnp.int32),
        pltpu.VMEM((NBUF, CHUNK, DP), jnp.float32),
        pltpu.SemaphoreType.DMA,
        pltpu.SemaphoreType.DMA,
    ],
)
def _gather_rows(idx_hbm, table_hbm, out_hbm, idx_v, rows_v, gsem, ssem):
    wid = lax.axis_index("s") * NC + lax.axis_index("c")
    b = wid // W_PER_B
    t0 = (wid % W_PER_B) * B_PER_W
    pltpu.sync_copy(idx_hbm.at[pl.ds(wid * B_PER_W, B_PER_W)], idx_v)

    gathers = [None] * NCHUNKS
    scatters = [None] * NCHUNKS
    s_waited = [False] * NCHUNKS
    for ch in range(min(NBUF, NCHUNKS)):
        gathers[ch] = pltpu.async_copy(
            table_hbm.at[idx_v.at[pl.ds(ch * CHUNK, CHUNK)]],
            rows_v.at[ch], gsem)
    for ch in range(NCHUNKS):
        gathers[ch].wait()
        scatters[ch] = pltpu.async_copy(
            rows_v.at[ch % NBUF],
            out_hbm.at[b, pl.ds(t0 + ch * CHUNK, CHUNK)], ssem)
        prev = ch - (NBUF - 1)
        nxt = ch + 1
        if prev >= 0 and nxt < NCHUNKS and gathers[nxt] is None:
            scatters[prev].wait()
            s_waited[prev] = True
            gathers[nxt] = pltpu.async_copy(
                table_hbm.at[idx_v.at[pl.ds(nxt * CHUNK, CHUNK)]],
                rows_v.at[nxt % NBUF], gsem)
    for ch in range(NCHUNKS):
        if not s_waited[ch]:
            scatters[ch].wait()


_TBLK = 512


def _transpose_body(half_ref, out_ref):
    t = half_ref[0].T
    out_ref[...] = t[None, :D, :]


def _transpose_body_acc(acc_ref, half_ref, out_ref):
    del acc_ref
    _transpose_body(half_ref, out_ref)


def _transpose_into(acc, half, h):
    grid = (B, TH // _TBLK)
    out_block = pl.BlockSpec(
        (1, D, _TBLK), lambda b, t, h=h: (b, 0, h * (TH // _TBLK) + t))
    half_block = pl.BlockSpec((1, _TBLK, DP), lambda b, t: (b, t, 0))
    if acc is None:
        return pl.pallas_call(
            _transpose_body,
            grid=grid,
            in_specs=[half_block],
            out_specs=out_block,
            out_shape=jax.ShapeDtypeStruct((B, D, T), jnp.float32),
        )(half)
    return pl.pallas_call(
        _transpose_body_acc,
        grid=grid,
        in_specs=[pl.BlockSpec(memory_space=pltpu.MemorySpace.HBM), half_block],
        out_specs=out_block,
        out_shape=jax.ShapeDtypeStruct((B, D, T), jnp.float32),
        input_output_aliases={0: 0},
    )(acc, half)


def kernel(idx, targets, table):
    del targets
    if idx.dtype != jnp.int32:
        idx = idx.astype(jnp.int32)
    table_pad = jnp.pad(table, ((0, 0), (0, DP - D)))
    halves = [
        _gather_rows(idx[:, h * TH:(h + 1) * TH].reshape(B * TH), table_pad)
        for h in range(NSPLIT)
    ]
    acc = None
    for h, half in enumerate(halves):
        acc = _transpose_into(acc, half, h)
    return acc.transpose(0, 2, 1)
